# baseline (speedup 1.0000x reference)
import functools

import jax
import jax.numpy as jnp
from jax import lax
from jax.experimental import pallas as pl
from jax.experimental.pallas import tpu as pltpu
from jax.experimental.pallas import tpu_sc as plsc

N = 10000
E = 160000
D_IN = 256
DH = 512
NCH = 4
CW = 128
NPAD = 10240
B = 128
RPT = 80
EPAD = 163840
NC = 2
NS = 16
HALF = 5120
AROWS = 5128
TRASH = 5127

_mesh = plsc.VectorSubcoreMesh(core_axis_name="c", subcore_axis_name="s")


def _deg_body(src_hbm, inv_hbm, idxv, onesv, zb, dbuf, ibuf, accum):
    c = lax.axis_index("c")
    s = lax.axis_index("s")
    w = s * NC + c

    for k in range(8):
        onesv[pl.ds(k * 16, 16)] = jnp.ones((16,), jnp.float32)
    for k in range(40):
        zb[pl.ds(k * 16, 16)] = jnp.zeros((16,), jnp.float32)

    pltpu.sync_copy(src_hbm.at[pl.ds(s * RPT, RPT)], idxv)
    pltpu.sync_copy(zb, accum.at[pl.ds(s * 640, 640)])
    plsc.subcore_barrier()

    def body(j, carry):
        pltpu.sync_copy(onesv, accum.at[idxv.at[j]], add=True)
        return carry

    lax.fori_loop(0, RPT, body, 0)
    plsc.subcore_barrier()

    pltpu.sync_copy(accum.at[pl.ds(w * 320, 320)], dbuf)
    for k in range(20):
        ibuf[pl.ds(k * 16, 16)] = 1.0 / (dbuf[pl.ds(k * 16, 16)] + 1e-16)
    pltpu.sync_copy(ibuf, inv_hbm.at[pl.ds(w * 320, 320)])


_deg_kernel = functools.partial(
    pl.kernel,
    mesh=_mesh,
    out_type=jax.ShapeDtypeStruct((NPAD,), jnp.float32),
    scratch_types=[
        pltpu.VMEM((RPT, B), jnp.int32),
        pltpu.VMEM((B,), jnp.float32),
        pltpu.VMEM((640,), jnp.float32),
        pltpu.VMEM((320,), jnp.float32),
        pltpu.VMEM((320,), jnp.float32),
        pltpu.VMEM_SHARED((NPAD,), jnp.float32),
    ],
)(_deg_body)


def _proj_body(x_ref, w_ref, b_ref, inv_ref, y_ref):
    z = jnp.dot(x_ref[...], w_ref[...], preferred_element_type=jnp.float32)
    y_ref[...] = (z + b_ref[...]) * inv_ref[...]


def _proj(x, W_V, b_V2, inv2):
    rb = 1000
    return pl.pallas_call(
        _proj_body,
        grid=(N // rb,),
        in_specs=[
            pl.BlockSpec((rb, D_IN), lambda i: (i, 0)),
            pl.BlockSpec((D_IN, DH), lambda i: (0, 0)),
            pl.BlockSpec((1, DH), lambda i: (0, 0)),
            pl.BlockSpec((rb, 1), lambda i: (i, 0)),
        ],
        out_specs=pl.BlockSpec((rb, DH), lambda i: (i, 0)),
        out_shape=jax.ShapeDtypeStruct((N, DH), jnp.float32),
    )(x, W_V, b_V2, inv2)


RPT0 = 16
RPT1 = 64
SEG = 16


def _scatter_body(ytab, src_hbm, dst_hbm, out_hbm,
                  isrc, idst, gb0, gb1, zb2, accum, sem):
    c = lax.axis_index("c")
    s = lax.axis_index("s")
    gb = (gb0, gb1)

    def zrow(i, carry):
        for k in range(CW // 16):
            zb2[i, pl.ds(k * 16, 16)] = jnp.zeros((16,), jnp.float32)
        return carry

    lax.fori_loop(0, 32, zrow, 0)

    rpt = jnp.where(c == 0, RPT0, RPT1)
    ebase = jnp.where(c == 0, s * RPT0, NS * RPT0 + s * RPT1)

    for ch in range(NCH):
        cds = pl.ds(ch * CW, CW)
        for r in range(20):
            pltpu.sync_copy(zb2, accum.at[pl.ds(s * 640 + r * 32, 32)])

        plsc.subcore_barrier()

        @pl.loop(0, rpt // SEG)
        def _(gseg):
            g0 = ebase + gseg * SEG
            pltpu.sync_copy(src_hbm.at[pl.ds(g0, SEG)], isrc)
            pltpu.sync_copy(dst_hbm.at[pl.ds(g0, SEG)], idst)

            pltpu.async_copy(ytab.at[isrc.at[0], cds], gb0, sem)
            pltpu.async_copy(ytab.at[isrc.at[1], cds], gb1, sem)

            @pl.loop(0, SEG - 2, step=2)
            def _(jj):
                for b in range(2):
                    j = jj + b
                    pltpu.make_async_copy(
                        ytab.at[pl.ds(0, B), cds], gb[b], sem).wait()
                    pltpu.sync_copy(gb[b], accum.at[idst.at[j]], add=True)
                    pltpu.async_copy(ytab.at[isrc.at[j + 2], cds], gb[b], sem)

            for b in range(2):
                pltpu.make_async_copy(
                    ytab.at[pl.ds(0, B), cds], gb[b], sem).wait()
                pltpu.sync_copy(gb[b], accum.at[idst.at[SEG - 2 + b]], add=True)

        plsc.subcore_barrier()
        pltpu.sync_copy(
            accum.at[pl.ds(s * 640, 640)],
            out_hbm.at[c, pl.ds(s * 640, 640), cds])
        plsc.subcore_barrier()


_scatter_kernel = functools.partial(
    pl.kernel,
    mesh=_mesh,
    out_type=jax.ShapeDtypeStruct((NC, NPAD, DH), jnp.float32),
    scratch_types=[
        pltpu.VMEM((SEG, B), jnp.int32),
        pltpu.VMEM((SEG, B), jnp.int32),
        pltpu.VMEM((B, CW), jnp.float32),
        pltpu.VMEM((B, CW), jnp.float32),
        pltpu.VMEM((32, CW), jnp.float32),
        pltpu.VMEM_SHARED((NPAD, CW), jnp.float32),
        pltpu.SemaphoreType.DMA,
    ],
)(_scatter_body)


def _ln(t, g, b):
    mu = jnp.mean(t, axis=-1, keepdims=True)
    var = jnp.mean((t - mu) ** 2, axis=-1, keepdims=True)
    return (t - mu) / jnp.sqrt(var + 1e-5) * g + b


def _post_body(oa_ref, ob_ref, att_ref, g0_ref, be0_ref, g1_ref, be1_ref,
               w1_ref, b1_ref, w2_ref, b2_ref, out_ref):
    t = oa_ref[0] + ob_ref[0] + att_ref[...]
    t = _ln(t, g0_ref[...], be0_ref[...])
    h = jnp.maximum(
        jnp.dot(t, w1_ref[...], preferred_element_type=jnp.float32) + b1_ref[...],
        0.0)
    f = jnp.dot(h, w2_ref[...], preferred_element_type=jnp.float32) + b2_ref[...]
    out_ref[...] = _ln(t + jnp.maximum(f, 0.0), g1_ref[...], be1_ref[...])


def _post(out0, att2, g0, be0, g1, be1, W1, b1, W2, b2):
    rb = 1000
    vec = lambda: pl.BlockSpec((1, DH), lambda i: (0, 0))
    return pl.pallas_call(
        _post_body,
        grid=(N // rb,),
        in_specs=[
            pl.BlockSpec((1, rb, DH), lambda i: (0, i, 0)),
            pl.BlockSpec((1, rb, DH), lambda i: (1, i, 0)),
            vec(), vec(), vec(), vec(), vec(),
            pl.BlockSpec((DH, DH), lambda i: (0, 0)),
            vec(),
            pl.BlockSpec((DH, DH), lambda i: (0, 0)),
            vec(),
        ],
        out_specs=pl.BlockSpec((rb, DH), lambda i: (i, 0)),
        out_shape=jax.ShapeDtypeStruct((N, DH), jnp.float32),
    )(out0, out0, att2, g0, be0, g1, be1, W1, b1, W2, b2)


def kernel(x, v2e_src, v2e_dst, W_K, b_K, W_V, b_V, att_r, W1, b1, W2, b2,
           g0, be0, g1, be1):
    npad = EPAD - E
    spread = N + (jnp.arange(npad, dtype=jnp.int32) % (NPAD - N))
    srcA = jnp.concatenate([v2e_src, spread]).reshape(EPAD // B, B)
    srcC = jnp.concatenate(
        [v2e_src, jnp.zeros((npad,), jnp.int32)]).reshape(EPAD // B, B)
    dstC = jnp.concatenate([v2e_dst, spread]).reshape(EPAD // B, B)

    inv = _deg_kernel(srcA)
    inv2 = inv[:N].reshape(N, 1)

    y = _proj(x, W_V, b_V.reshape(1, DH), inv2)

    out0 = _scatter_kernel(y, srcC, dstC)

    out = _post(out0, att_r.reshape(1, DH),
                g0.reshape(1, DH), be0.reshape(1, DH),
                g1.reshape(1, DH), be1.reshape(1, DH),
                W1, b1.reshape(1, DH), W2, b2.reshape(1, DH))
    return out


# speedup vs baseline: 1.2282x; 1.2282x over previous
import functools

import jax
import jax.numpy as jnp
from jax import lax
from jax.experimental import pallas as pl
from jax.experimental.pallas import tpu as pltpu
from jax.experimental.pallas import tpu_sc as plsc

N = 10000
E = 160000
D_IN = 256
DH = 512
NCH = 4
CW = 128
NPAD = 10240
B = 128
RPT = 80
EPAD = 163840
NC = 2
NS = 16
HALF = 5120
AROWS = 5128
TRASH = 5127

_mesh = plsc.VectorSubcoreMesh(core_axis_name="c", subcore_axis_name="s")


def _deg_body(src_hbm, inv_hbm, idxv, onesv, zb, dbuf, ibuf, accum):
    c = lax.axis_index("c")
    s = lax.axis_index("s")
    w = s * NC + c

    for k in range(8):
        onesv[pl.ds(k * 16, 16)] = jnp.ones((16,), jnp.float32)
    for k in range(40):
        zb[pl.ds(k * 16, 16)] = jnp.zeros((16,), jnp.float32)

    pltpu.sync_copy(src_hbm.at[pl.ds(s * RPT, RPT)], idxv)
    pltpu.sync_copy(zb, accum.at[pl.ds(s * 640, 640)])
    plsc.subcore_barrier()

    def body(j, carry):
        pltpu.sync_copy(onesv, accum.at[idxv.at[j]], add=True)
        return carry

    lax.fori_loop(0, RPT, body, 0)
    plsc.subcore_barrier()

    pltpu.sync_copy(accum.at[pl.ds(w * 320, 320)], dbuf)
    for k in range(20):
        ibuf[pl.ds(k * 16, 16)] = 1.0 / (dbuf[pl.ds(k * 16, 16)] + 1e-16)
    pltpu.sync_copy(ibuf, inv_hbm.at[pl.ds(w * 320, 320)])


_deg_kernel = functools.partial(
    pl.kernel,
    mesh=_mesh,
    out_type=jax.ShapeDtypeStruct((NPAD,), jnp.float32),
    scratch_types=[
        pltpu.VMEM((RPT, B), jnp.int32),
        pltpu.VMEM((B,), jnp.float32),
        pltpu.VMEM((640,), jnp.float32),
        pltpu.VMEM((320,), jnp.float32),
        pltpu.VMEM((320,), jnp.float32),
        pltpu.VMEM_SHARED((NPAD,), jnp.float32),
    ],
)(_deg_body)


def _proj_body(x_ref, w_ref, b_ref, inv_ref, y_ref):
    z = jnp.dot(x_ref[...], w_ref[...], preferred_element_type=jnp.float32)
    y_ref[...] = (z + b_ref[...]) * inv_ref[...]


def _proj(x, W_V, b_V2, inv2):
    rb = 1000
    return pl.pallas_call(
        _proj_body,
        grid=(N // rb,),
        in_specs=[
            pl.BlockSpec((rb, D_IN), lambda i: (i, 0)),
            pl.BlockSpec((D_IN, DH), lambda i: (0, 0)),
            pl.BlockSpec((1, DH), lambda i: (0, 0)),
            pl.BlockSpec((rb, 1), lambda i: (i, 0)),
        ],
        out_specs=pl.BlockSpec((rb, DH), lambda i: (i, 0)),
        out_shape=jax.ShapeDtypeStruct((N, DH), jnp.float32),
    )(x, W_V, b_V2, inv2)


RPT0 = 64
RPT1 = 16
SEG = 16


def _scatter_body(ytab, src_hbm, dst_hbm, out_hbm,
                  isrc, idst, gb0, gb1, zb2, accum, sem):
    c = lax.axis_index("c")
    s = lax.axis_index("s")
    gb = (gb0, gb1)

    def zrow(i, carry):
        for k in range(CW // 16):
            zb2[i, pl.ds(k * 16, 16)] = jnp.zeros((16,), jnp.float32)
        return carry

    lax.fori_loop(0, 32, zrow, 0)

    rpt = jnp.where(c == 0, RPT0, RPT1)
    ebase = jnp.where(c == 0, s * RPT0, NS * RPT0 + s * RPT1)

    for ch in range(NCH):
        cds = pl.ds(ch * CW, CW)
        for r in range(20):
            pltpu.sync_copy(zb2, accum.at[pl.ds(s * 640 + r * 32, 32)])

        plsc.subcore_barrier()

        @pl.loop(0, rpt // SEG)
        def _(gseg):
            g0 = ebase + gseg * SEG
            pltpu.sync_copy(src_hbm.at[pl.ds(g0, SEG)], isrc)
            pltpu.sync_copy(dst_hbm.at[pl.ds(g0, SEG)], idst)

            pltpu.async_copy(ytab.at[isrc.at[0], cds], gb0, sem)
            pltpu.async_copy(ytab.at[isrc.at[1], cds], gb1, sem)

            @pl.loop(0, SEG - 2, step=2)
            def _(jj):
                for b in range(2):
                    j = jj + b
                    pltpu.make_async_copy(
                        ytab.at[pl.ds(0, B), cds], gb[b], sem).wait()
                    pltpu.sync_copy(gb[b], accum.at[idst.at[j]], add=True)
                    pltpu.async_copy(ytab.at[isrc.at[j + 2], cds], gb[b], sem)

            for b in range(2):
                pltpu.make_async_copy(
                    ytab.at[pl.ds(0, B), cds], gb[b], sem).wait()
                pltpu.sync_copy(gb[b], accum.at[idst.at[SEG - 2 + b]], add=True)

        plsc.subcore_barrier()
        pltpu.sync_copy(
            accum.at[pl.ds(s * 640, 640)],
            out_hbm.at[c, pl.ds(s * 640, 640), cds])
        plsc.subcore_barrier()


_scatter_kernel = functools.partial(
    pl.kernel,
    mesh=_mesh,
    out_type=jax.ShapeDtypeStruct((NC, NPAD, DH), jnp.float32),
    scratch_types=[
        pltpu.VMEM((SEG, B), jnp.int32),
        pltpu.VMEM((SEG, B), jnp.int32),
        pltpu.VMEM((B, CW), jnp.float32),
        pltpu.VMEM((B, CW), jnp.float32),
        pltpu.VMEM((32, CW), jnp.float32),
        pltpu.VMEM_SHARED((NPAD, CW), jnp.float32),
        pltpu.SemaphoreType.DMA,
    ],
)(_scatter_body)


def _ln(t, g, b):
    mu = jnp.mean(t, axis=-1, keepdims=True)
    var = jnp.mean((t - mu) ** 2, axis=-1, keepdims=True)
    return (t - mu) / jnp.sqrt(var + 1e-5) * g + b


def _post_body(oa_ref, ob_ref, att_ref, g0_ref, be0_ref, g1_ref, be1_ref,
               w1_ref, b1_ref, w2_ref, b2_ref, out_ref):
    t = oa_ref[0] + ob_ref[0] + att_ref[...]
    t = _ln(t, g0_ref[...], be0_ref[...])
    h = jnp.maximum(
        jnp.dot(t, w1_ref[...], preferred_element_type=jnp.float32) + b1_ref[...],
        0.0)
    f = jnp.dot(h, w2_ref[...], preferred_element_type=jnp.float32) + b2_ref[...]
    out_ref[...] = _ln(t + jnp.maximum(f, 0.0), g1_ref[...], be1_ref[...])


def _post(out0, att2, g0, be0, g1, be1, W1, b1, W2, b2):
    rb = 1000
    vec = lambda: pl.BlockSpec((1, DH), lambda i: (0, 0))
    return pl.pallas_call(
        _post_body,
        grid=(N // rb,),
        in_specs=[
            pl.BlockSpec((1, rb, DH), lambda i: (0, i, 0)),
            pl.BlockSpec((1, rb, DH), lambda i: (1, i, 0)),
            vec(), vec(), vec(), vec(), vec(),
            pl.BlockSpec((DH, DH), lambda i: (0, 0)),
            vec(),
            pl.BlockSpec((DH, DH), lambda i: (0, 0)),
            vec(),
        ],
        out_specs=pl.BlockSpec((rb, DH), lambda i: (i, 0)),
        out_shape=jax.ShapeDtypeStruct((N, DH), jnp.float32),
    )(out0, out0, att2, g0, be0, g1, be1, W1, b1, W2, b2)


def kernel(x, v2e_src, v2e_dst, W_K, b_K, W_V, b_V, att_r, W1, b1, W2, b2,
           g0, be0, g1, be1):
    npad = EPAD - E
    spread = N + (jnp.arange(npad, dtype=jnp.int32) % (NPAD - N))
    srcA = jnp.concatenate([v2e_src, spread]).reshape(EPAD // B, B)
    srcC = jnp.concatenate(
        [v2e_src, jnp.zeros((npad,), jnp.int32)]).reshape(EPAD // B, B)
    dstC = jnp.concatenate([v2e_dst, spread]).reshape(EPAD // B, B)

    inv = _deg_kernel(srcA)
    inv2 = inv[:N].reshape(N, 1)

    y = _proj(x, W_V, b_V.reshape(1, DH), inv2)

    out0 = _scatter_kernel(y, srcC, dstC)

    out = _post(out0, att_r.reshape(1, DH),
                g0.reshape(1, DH), be0.reshape(1, DH),
                g1.reshape(1, DH), be1.reshape(1, DH),
                W1, b1.reshape(1, DH), W2, b2.reshape(1, DH))
    return out
